# Initial kernel scaffold; baseline (speedup 1.0000x reference)
import jax
import jax.numpy as jnp
from jax.experimental import pallas as pl


def kernel(input_ids, attention_mask, acoustic_hidden, embed_table, W1, b1, W2, b2):
    raise NotImplementedError("write your pallas kernel here")


# speedup vs baseline: 3.4884x; 3.4884x over previous
import functools

import numpy as np
import jax
import jax.numpy as jnp
from jax import lax
from jax.experimental import pallas as pl
from jax.experimental.pallas import tpu as pltpu
from jax.experimental.pallas import tpu_sc as plsc

_AUDIO_TOKEN_ID = 32000
_D = 4096
_B = 4
_S = 2048
_NF = 10
_TA = 249
_AC = 1024
_OUT_S = _S + _NF - 1
_ROWS = _B * _OUT_S
_PAIRS = _B * _S

_NC, _NS = 2, 16
_NW = _NC * _NS
_K = 8
_NCH = _PAIRS // (_NW * _K)


def _mlp_body(ac_ref, w1_ref, b1_ref, w2_ref, b2_ref, out_ref, h_ref):
    j = pl.program_id(0)

    @pl.when(j == 0)
    def _first():
        pos = np.arange(_NF, dtype=np.float32) * np.float32(
            (_TA - 1) / (_NF - 1))
        lo = np.floor(pos).astype(np.int32)
        hi = np.minimum(lo + 1, _TA - 1)
        frac = (pos - lo.astype(np.float32)).astype(np.float32)
        rows = []
        for k in range(_NF):
            r = (ac_ref[:, int(lo[k]), :] * (1.0 - float(frac[k]))
                 + ac_ref[:, int(hi[k]), :] * float(frac[k]))
            rows.append(r)
        acx = jnp.concatenate(rows, axis=0)
        h = jnp.dot(acx, w1_ref[...],
                    preferred_element_type=jnp.float32) + b1_ref[...]
        h_ref[...] = h * (1.0 / (1.0 + jnp.exp(-h)))

    out_ref[...] = jnp.dot(h_ref[...], w2_ref[...],
                           preferred_element_type=jnp.float32) + b2_ref[...]


def _audio_mlp(acoustic_hidden, W1, b1, W2, b2):
    ncol = 8
    blk = _D // ncol
    return pl.pallas_call(
        _mlp_body,
        grid=(ncol,),
        in_specs=[
            pl.BlockSpec((_B, _TA, _AC), lambda j: (0, 0, 0)),
            pl.BlockSpec((_AC, _D), lambda j: (0, 0)),
            pl.BlockSpec((1, _D), lambda j: (0, 0)),
            pl.BlockSpec((_D, blk), lambda j: (0, j)),
            pl.BlockSpec((1, blk), lambda j: (0, j)),
        ],
        out_specs=pl.BlockSpec((_NF * _B, blk), lambda j: (0, j)),
        out_shape=jax.ShapeDtypeStruct((_NF * _B, _D), jnp.float32),
        scratch_shapes=[pltpu.VMEM((_NF * _B, _D), jnp.float32)],
    )(acoustic_hidden, W1, b1.reshape(1, _D), W2, b2.reshape(1, _D))


def _sc_merge(embed_table, audio_rows, src_ids, dst_ids, a_src, a_dst):
    mesh = plsc.VectorSubcoreMesh(core_axis_name="c", subcore_axis_name="s")

    @functools.partial(
        pl.kernel,
        mesh=mesh,
        out_type=jax.ShapeDtypeStruct((_ROWS, _D), jnp.float32),
        scratch_types=[
            pltpu.VMEM((_NCH, _K), jnp.int32),
            pltpu.VMEM((_NCH, _K), jnp.int32),
            pltpu.VMEM((2, _K), jnp.int32),
            pltpu.VMEM((2, _K), jnp.int32),
            pltpu.VMEM((_K, _D), jnp.float32),
            pltpu.SemaphoreType.DMA,
            pltpu.SemaphoreType.DMA,
        ],
    )
    def merge(table_hbm, audio_hbm, src_hbm, dst_hbm, asrc_hbm, adst_hbm,
              out_hbm, src_v, dst_v, asrc_v, adst_v, buf, sem_g, sem_s):
        wid = lax.axis_index("s") * _NC + lax.axis_index("c")
        pltpu.sync_copy(src_hbm.at[wid], src_v)
        pltpu.sync_copy(dst_hbm.at[wid], dst_v)

        def chunk(c, carry):
            pltpu.async_copy(table_hbm.at[src_v.at[c]], buf, sem_g).wait()
            pltpu.async_copy(buf, out_hbm.at[dst_v.at[c]], sem_s).wait()
            return carry

        lax.fori_loop(0, _NCH, chunk, 0)

        @pl.when(wid < _B)
        def _audio():
            pltpu.sync_copy(asrc_hbm.at[wid], asrc_v)
            pltpu.sync_copy(adst_hbm.at[wid], adst_v)

            def achunk(c, carry):
                pltpu.async_copy(audio_hbm.at[asrc_v.at[c]], buf,
                                 sem_g).wait()
                pltpu.async_copy(buf, out_hbm.at[adst_v.at[c]],
                                 sem_s).wait()
                return carry

            lax.fori_loop(0, 2, achunk, 0)

    return merge(embed_table, audio_rows, src_ids, dst_ids, a_src, a_dst)


def kernel(input_ids, attention_mask, acoustic_hidden, embed_table,
           W1, b1, W2, b2):
    p = jnp.argmax(input_ids == _AUDIO_TOKEN_ID, axis=1).astype(jnp.int32)
    i = jnp.arange(_S, dtype=jnp.int32)[None, :]
    pc = p[:, None]
    dst_local = i + jnp.where(i > pc, 9, 0)
    is_audio = i == pc
    dup = jnp.where(p == 0, 1, 0).astype(jnp.int32)[:, None]
    dup_tok = jnp.take_along_axis(input_ids, dup, axis=1)
    src_tok = jnp.where(is_audio, dup_tok, input_ids)
    dst_local = jnp.where(is_audio, dup + jnp.where(dup > pc, 9, 0),
                          dst_local)
    base = (jnp.arange(_B, dtype=jnp.int32) * _OUT_S)[:, None]
    dst_rows = dst_local + base
    src_ids = src_tok.astype(jnp.int32).reshape(_NW, _NCH, _K)
    dst_ids = dst_rows.astype(jnp.int32).reshape(_NW, _NCH, _K)

    k16 = jnp.minimum(jnp.arange(2 * _K, dtype=jnp.int32), _NF - 1)[None, :]
    bidx = jnp.arange(_B, dtype=jnp.int32)[:, None]
    a_src = (k16 * _B + bidx).reshape(_B, 2, _K)
    a_dst = (bidx * _OUT_S + p[:, None] + k16).reshape(_B, 2, _K)

    audio_rows = _audio_mlp(acoustic_hidden, W1, b1, W2, b2)

    flat = _sc_merge(embed_table, audio_rows, src_ids, dst_ids, a_src, a_dst)
    final_embedding = flat.reshape(_B, _OUT_S, _D)

    am = attention_mask.astype(jnp.int32)
    dup_am = jnp.take_along_axis(am, dup, axis=1)
    am_vals = jnp.where(is_audio, dup_am, am)
    fam = jnp.zeros((_B, _OUT_S), jnp.int32)
    fam = fam.at[bidx, dst_local].set(am_vals)
    audio_pos = p[:, None] + jnp.arange(_NF, dtype=jnp.int32)[None, :]
    fam = fam.at[bidx, audio_pos].set(1)
    position_ids = jnp.cumsum(fam, axis=-1) - 1
    position_ids = jnp.where(fam == 0, 1, position_ids).astype(jnp.int32)
    return final_embedding, fam, position_ids
